# Initial kernel scaffold; baseline (speedup 1.0000x reference)
import functools

import jax
import jax.numpy as jnp
from jax import lax
from jax.experimental import pallas as pl
from jax.experimental.pallas import tpu as pltpu
from jax.experimental.pallas import tpu_sc as plsc

_B0, _B1 = 4096, 50
_D = 128
_B = _B0 * _B1
_NC, _NS = 2, 16
_NW = _NC * _NS
_BPW = _B // _NW
_CHUNK = 128
_NCH = _BPW // _CHUNK

_mesh = plsc.VectorSubcoreMesh(core_axis_name="c", subcore_axis_name="s")


@functools.partial(
    pl.kernel,
    mesh=_mesh,
    out_type=jax.ShapeDtypeStruct((_B, _D), jnp.float32),
    scratch_types=[
        pltpu.VMEM((_NCH, _CHUNK), jnp.int32),
        pltpu.VMEM((_CHUNK, _D), jnp.float32),
        pltpu.SemaphoreType.DMA,
    ],
)
def _emb_lookup(table_hbm, idx_hbm, out_hbm, idx_v, rows_v, sem):
    wid = lax.axis_index("s") * _NC + lax.axis_index("c")
    base = wid * _BPW
    pltpu.sync_copy(idx_hbm.at[pl.ds(wid * _NCH, _NCH)], idx_v)

    @pl.loop(0, _NCH)
    def _chunk(j):
        pltpu.async_copy(table_hbm.at[idx_v.at[j]], rows_v, sem).wait()
        pltpu.sync_copy(rows_v, out_hbm.at[pl.ds(base + j * _CHUNK, _CHUNK)])


def kernel(token_ids, weight):
    idx2d = token_ids.reshape(_NW * _NCH, _CHUNK).astype(jnp.int32)
    out = _emb_lookup(weight, idx2d)
    return out.reshape(_B0, _B1, _D)


# speedup vs baseline: 2.9669x; 2.9669x over previous
import functools

import jax
import jax.numpy as jnp
from jax import lax
from jax.experimental import pallas as pl
from jax.experimental.pallas import tpu as pltpu
from jax.experimental.pallas import tpu_sc as plsc

_B0, _B1 = 4096, 50
_D = 128
_B = _B0 * _B1
_NC, _NS = 2, 16
_NW = _NC * _NS
_BPW = _B // _NW
_CHUNK = 128
_NCH = _BPW // _CHUNK

_mesh = plsc.VectorSubcoreMesh(core_axis_name="c", subcore_axis_name="s")


@functools.partial(
    pl.kernel,
    mesh=_mesh,
    out_type=jax.ShapeDtypeStruct((_B, _D), jnp.float32),
    scratch_types=[
        pltpu.VMEM((_BPW,), jnp.int32),
        pltpu.VMEM((_CHUNK, _D), jnp.float32),
        pltpu.SemaphoreType.DMA,
    ],
)
def _emb_lookup(table_hbm, idx_hbm, out_hbm, idx_v, rows_v, sem):
    wid = lax.axis_index("s") * _NC + lax.axis_index("c")
    base = wid * _BPW
    pltpu.sync_copy(idx_hbm.at[pl.ds(base, _BPW)], idx_v)

    @pl.loop(0, _NCH)
    def _chunk(j):
        idx_c = idx_v.at[pl.ds(j * _CHUNK, _CHUNK)]
        pltpu.async_copy(table_hbm.at[idx_c], rows_v, sem).wait()
        pltpu.sync_copy(rows_v, out_hbm.at[pl.ds(base + j * _CHUNK, _CHUNK)])


def kernel(token_ids, weight):
    idx1d = token_ids.reshape(_B).astype(jnp.int32)
    out = _emb_lookup(weight, idx1d)
    return out.reshape(_B0, _B1, _D)


# speedup vs baseline: 3.3350x; 1.1241x over previous
import functools

import jax
import jax.numpy as jnp
from jax import lax
from jax.experimental import pallas as pl
from jax.experimental.pallas import tpu as pltpu
from jax.experimental.pallas import tpu_sc as plsc

_B0, _B1 = 4096, 50
_D = 128
_B = _B0 * _B1
_NC, _NS = 2, 16
_NW = _NC * _NS
_BPW = _B // _NW
_CHUNK = 128
_NCH = _BPW // _CHUNK
_NBUF = 5

_mesh = plsc.VectorSubcoreMesh(core_axis_name="c", subcore_axis_name="s")


@functools.partial(
    pl.kernel,
    mesh=_mesh,
    out_type=jax.ShapeDtypeStruct((_B, _D), jnp.float32),
    scratch_types=[
        pltpu.VMEM((_BPW,), jnp.int32),
        pltpu.VMEM((_NBUF, _CHUNK, _D), jnp.float32),
        [pltpu.SemaphoreType.DMA] * _NBUF,
        [pltpu.SemaphoreType.DMA] * _NBUF,
    ],
)
def _emb_lookup(table_hbm, idx_hbm, out_hbm, idx_v, rows_v, gsem, ssem):
    wid = lax.axis_index("s") * _NC + lax.axis_index("c")
    base = wid * _BPW
    pltpu.sync_copy(idx_hbm.at[pl.ds(base, _BPW)], idx_v)

    def start_g(j, b):
        idx_c = idx_v.at[pl.ds(j * _CHUNK, _CHUNK)]
        pltpu.async_copy(table_hbm.at[idx_c], rows_v.at[b], gsem[b])

    def wait_g(b):
        pltpu.make_async_copy(
            table_hbm.at[pl.ds(0, _CHUNK)], rows_v.at[b], gsem[b]).wait()

    def start_s(j, b):
        pltpu.async_copy(
            rows_v.at[b], out_hbm.at[pl.ds(base + j * _CHUNK, _CHUNK)], ssem[b])

    def wait_s(b):
        pltpu.make_async_copy(
            rows_v.at[b], out_hbm.at[pl.ds(base, _CHUNK)], ssem[b]).wait()

    for b in range(_NBUF - 1):
        start_g(b, b)
    wait_g(0)
    start_s(0, 0)
    start_g(_NBUF - 1, _NBUF - 1)

    @pl.loop(1, _NCH - _NBUF + 1, step=_NBUF)
    def _steady(i):
        for u in range(_NBUF):
            j = i + u
            b = (u + 1) % _NBUF
            bp = (u) % _NBUF
            wait_g(b)
            start_s(j, b)
            wait_s(bp)
            start_g(j + _NBUF - 1, bp)

    for j in range(_NCH - _NBUF + 1, _NCH):
        b = j % _NBUF
        wait_g(b)
        start_s(j, b)
    for j in range(_NCH - _NBUF, _NCH):
        wait_s(j % _NBUF)


def kernel(token_ids, weight):
    idx1d = token_ids.reshape(_B).astype(jnp.int32)
    out = _emb_lookup(weight, idx1d)
    return out.reshape(_B0, _B1, _D)


# speedup vs baseline: 5.9712x; 1.7905x over previous
import functools

import jax
import jax.numpy as jnp
from jax import lax
from jax.experimental import pallas as pl
from jax.experimental.pallas import tpu as pltpu
from jax.experimental.pallas import tpu_sc as plsc

_S, _T = 4096, 50
_D = 128
_NC, _NS = 2, 16
_NW = _NC * _NS
_SPW = _S // _NW
_CS = 4
_NCH = _SPW // _CS
_NBUF = 4

_mesh = plsc.VectorSubcoreMesh(core_axis_name="c", subcore_axis_name="s")


@functools.partial(
    pl.kernel,
    mesh=_mesh,
    out_type=jax.ShapeDtypeStruct((_S, _T, _D), jnp.float32),
    scratch_types=[
        pltpu.VMEM((_SPW, _T), jnp.int32),
        pltpu.VMEM((_NBUF, _CS, _T, _D), jnp.float32),
        [pltpu.SemaphoreType.DMA] * _NBUF,
        [pltpu.SemaphoreType.DMA] * _NBUF,
    ],
)
def _emb_lookup(table_hbm, idx_hbm, out_hbm, idx_v, rows_v, gsem, ssem):
    wid = lax.axis_index("s") * _NC + lax.axis_index("c")
    s0 = wid * _SPW
    pltpu.sync_copy(idx_hbm.at[pl.ds(s0, _SPW)], idx_v)

    def start_g(j, b):
        for i in range(_CS):
            pltpu.async_copy(
                table_hbm.at[idx_v.at[j * _CS + i]], rows_v.at[b, i], gsem[b])

    def wait_g(b):
        for i in range(_CS):
            pltpu.make_async_copy(
                table_hbm.at[idx_v.at[0]], rows_v.at[b, i], gsem[b]).wait()

    def start_s(j, b):
        pltpu.async_copy(
            rows_v.at[b], out_hbm.at[pl.ds(s0 + j * _CS, _CS)], ssem[b])

    def wait_s(b):
        pltpu.make_async_copy(
            rows_v.at[b], out_hbm.at[pl.ds(0, _CS)], ssem[b]).wait()

    for b in range(_NBUF - 1):
        start_g(b, b)
    wait_g(0)
    start_s(0, 0)
    start_g(_NBUF - 1, _NBUF - 1)

    @pl.loop(1, _NCH - _NBUF + 1, step=_NBUF)
    def _steady(i):
        for u in range(_NBUF):
            j = i + u
            b = (u + 1) % _NBUF
            bp = u % _NBUF
            wait_g(b)
            start_s(j, b)
            wait_s(bp)
            start_g(j + _NBUF - 1, bp)

    for j in range(_NCH - _NBUF + 1, _NCH):
        b = j % _NBUF
        wait_g(b)
        start_s(j, b)
    for j in range(_NCH - _NBUF, _NCH):
        wait_s(j % _NBUF)


def kernel(token_ids, weight):
    return _emb_lookup(weight, token_ids.astype(jnp.int32))


# speedup vs baseline: 10.7479x; 1.7999x over previous
import functools

import jax
import jax.numpy as jnp
from jax import lax
from jax.experimental import pallas as pl
from jax.experimental.pallas import tpu as pltpu
from jax.experimental.pallas import tpu_sc as plsc

_S, _T = 4096, 50
_D = 128
_NC, _NS = 2, 16
_NW = _NC * _NS
_SPW = _S // _NW
_NCH = _T
_NBUF = 5

_mesh = plsc.VectorSubcoreMesh(core_axis_name="c", subcore_axis_name="s")


@functools.partial(
    pl.kernel,
    mesh=_mesh,
    out_type=jax.ShapeDtypeStruct((_T, _S, _D), jnp.float32),
    scratch_types=[
        pltpu.VMEM((_T, _SPW), jnp.int32),
        pltpu.VMEM((_NBUF, _SPW, _D), jnp.float32),
        [pltpu.SemaphoreType.DMA] * _NBUF,
        [pltpu.SemaphoreType.DMA] * _NBUF,
    ],
)
def _emb_lookup(table_hbm, idx_hbm, out_hbm, idx_v, rows_v, gsem, ssem):
    wid = lax.axis_index("s") * _NC + lax.axis_index("c")
    s0 = wid * _SPW
    pltpu.sync_copy(idx_hbm.at[:, pl.ds(s0, _SPW)], idx_v)

    def start_g(j, b):
        pltpu.async_copy(table_hbm.at[idx_v.at[j]], rows_v.at[b], gsem[b])

    def wait_g(b):
        pltpu.make_async_copy(
            table_hbm.at[idx_v.at[0]], rows_v.at[b], gsem[b]).wait()

    def start_s(j, b):
        pltpu.async_copy(
            rows_v.at[b], out_hbm.at[j, pl.ds(s0, _SPW)], ssem[b])

    def wait_s(b):
        pltpu.make_async_copy(
            rows_v.at[b], out_hbm.at[0, pl.ds(s0, _SPW)], ssem[b]).wait()

    for b in range(_NBUF - 1):
        start_g(b, b)
    wait_g(0)
    start_s(0, 0)
    start_g(_NBUF - 1, _NBUF - 1)

    @pl.loop(1, _NCH - _NBUF + 1, step=_NBUF)
    def _steady(i):
        for u in range(_NBUF):
            j = i + u
            b = (u + 1) % _NBUF
            bp = u % _NBUF
            wait_g(b)
            start_s(j, b)
            wait_s(bp)
            start_g(j + _NBUF - 1, bp)

    for j in range(_NCH - _NBUF + 1, _NCH):
        b = j % _NBUF
        wait_g(b)
        start_s(j, b)
    for j in range(_NCH - _NBUF, _NCH):
        wait_s(j % _NBUF)


def kernel(token_ids, weight):
    idx_t = jnp.transpose(token_ids).astype(jnp.int32)
    out_t = _emb_lookup(weight, idx_t)
    return jnp.transpose(out_t, (1, 0, 2))
